# baseline (speedup 1.0000x reference)
import functools

import jax
import jax.numpy as jnp
from jax import lax
from jax.experimental import pallas as pl
from jax.experimental.pallas import tpu as pltpu
from jax.experimental.pallas import tpu_sc as plsc

EPS = 1e-5
_CHUNK = 128


def _make_sc_gather(n_rows: int, d: int):
    info = plsc.get_sparse_core_info()
    nc, ns = info.num_cores, info.num_subcores
    nw = nc * ns
    assert n_rows % nw == 0
    b_per_w = n_rows // nw
    offs = list(range(0, b_per_w, _CHUNK))
    szs = [min(_CHUNK, b_per_w - o) for o in offs]
    n = len(offs)
    mesh = plsc.VectorSubcoreMesh(core_axis_name="c", subcore_axis_name="s")

    @functools.partial(
        pl.kernel,
        mesh=mesh,
        out_type=jax.ShapeDtypeStruct((n_rows, d), jnp.float32),
        scratch_types=[
            pltpu.VMEM((b_per_w,), jnp.int32),
            pltpu.VMEM((_CHUNK, d), jnp.float32),
            pltpu.VMEM((_CHUNK, d), jnp.float32),
            pltpu.SemaphoreType.DMA,
            pltpu.SemaphoreType.DMA,
            pltpu.SemaphoreType.DMA,
            pltpu.SemaphoreType.DMA,
        ],
    )
    def gather_kernel(table_hbm, idx_hbm, out_hbm, idx_v, buf_a, buf_b,
                      gs_a, gs_b, os_a, os_b):
        wid = lax.axis_index("s") * nc + lax.axis_index("c")
        base = wid * b_per_w
        pltpu.sync_copy(idx_hbm.at[pl.ds(base, b_per_w)], idx_v)
        bufs = (buf_a, buf_b)
        gsems = (gs_a, gs_b)
        osems = (os_a, os_b)

        def start_gather(k):
            i = k % 2
            h = pltpu.make_async_copy(
                table_hbm.at[idx_v.at[pl.ds(offs[k], szs[k])]],
                bufs[i].at[pl.ds(0, szs[k])],
                gsems[i])
            h.start()
            return h

        def start_out(k):
            i = k % 2
            h = pltpu.make_async_copy(
                bufs[i].at[pl.ds(0, szs[k])],
                out_hbm.at[pl.ds(base + offs[k], szs[k])],
                osems[i])
            h.start()
            return h

        g = {0: start_gather(0)}
        o = {}
        for k in range(n):
            if k + 1 < n:
                if k - 1 in o:
                    o[k - 1].wait()
                g[k + 1] = start_gather(k + 1)
            g[k].wait()
            o[k] = start_out(k)
        o[n - 1].wait()
        if n >= 2:
            o[n - 2].wait()

    return gather_kernel


def _ln_body(gath_ref, seg_ref, pos_ref, segtab_ref, gamma_ref, beta_ref,
             out_ref):
    x = gath_ref[...]
    sid = seg_ref[...].astype(jnp.float32)
    pos = pos_ref[...]
    st = segtab_ref[...]
    seg = st[0][None, None, :] + sid[:, :, None] * (st[1] - st[0])[None, None, :]
    comb = x + pos[:, None, :] + seg
    mean = jnp.mean(comb, axis=-1, keepdims=True)
    var = jnp.mean(jnp.square(comb - mean), axis=-1, keepdims=True)
    xhat = (comb - mean) * lax.rsqrt(var + EPS)
    out_ref[...] = (xhat * gamma_ref[...][None, None, :]
                    + beta_ref[...][None, None, :])


def _ln_call(gath3, seg_ids, pos, seg_table, gamma, beta, b_blk=128):
    l, b, d = gath3.shape
    grid = (b // b_blk,)
    return pl.pallas_call(
        _ln_body,
        grid=grid,
        in_specs=[
            pl.BlockSpec((l, b_blk, d), lambda i: (0, i, 0)),
            pl.BlockSpec((l, b_blk), lambda i: (0, i)),
            pl.BlockSpec((l, d), lambda i: (0, 0)),
            pl.BlockSpec((2, d), lambda i: (0, 0)),
            pl.BlockSpec((d,), lambda i: (0,)),
            pl.BlockSpec((d,), lambda i: (0,)),
        ],
        out_specs=pl.BlockSpec((l, b_blk, d), lambda i: (0, i, 0)),
        out_shape=jax.ShapeDtypeStruct((l, b, d), jnp.float32),
    )(gath3, seg_ids, pos, seg_table, gamma, beta)


def kernel(tokens, segment_ids, token_table, pos_table, seg_table, gamma,
           beta):
    b, l = tokens.shape
    d = token_table.shape[1]
    l0 = l // 2
    flat_t = tokens.swapaxes(0, 1).astype(jnp.int32)
    seg_t = segment_ids.swapaxes(0, 1).astype(jnp.int32)
    gather_fn = _make_sc_gather(l0 * b, d)
    outs = []
    for s in range(2):
        sl = slice(s * l0, (s + 1) * l0)
        gath = gather_fn(token_table, flat_t[sl].reshape(-1)).reshape(l0, b, d)
        outs.append(_ln_call(gath, seg_t[sl], pos_table[sl], seg_table,
                             gamma, beta))
    return jnp.concatenate(outs, axis=0).swapaxes(0, 1)


# speedup vs baseline: 1.2928x; 1.2928x over previous
import functools

import jax
import jax.numpy as jnp
from jax import lax
from jax.experimental import pallas as pl
from jax.experimental.pallas import tpu as pltpu
from jax.experimental.pallas import tpu_sc as plsc

EPS = 1e-5
_CHUNK = 128


def _make_sc_gather(n_rows: int, d: int):
    info = plsc.get_sparse_core_info()
    nc, ns = info.num_cores, info.num_subcores
    nw = nc * ns
    assert n_rows % nw == 0
    b_per_w = n_rows // nw
    offs = list(range(0, b_per_w, _CHUNK))
    szs = [min(_CHUNK, b_per_w - o) for o in offs]
    n = len(offs)
    mesh = plsc.VectorSubcoreMesh(core_axis_name="c", subcore_axis_name="s")

    @functools.partial(
        pl.kernel,
        mesh=mesh,
        out_type=jax.ShapeDtypeStruct((n_rows, d), jnp.float32),
        scratch_types=[
            pltpu.VMEM((b_per_w,), jnp.int32),
            pltpu.VMEM((_CHUNK, d), jnp.float32),
            pltpu.VMEM((_CHUNK, d), jnp.float32),
            pltpu.SemaphoreType.DMA,
            pltpu.SemaphoreType.DMA,
            pltpu.SemaphoreType.DMA,
            pltpu.SemaphoreType.DMA,
        ],
    )
    def gather_kernel(table_hbm, idx_hbm, out_hbm, idx_v, buf_a, buf_b,
                      gs_a, gs_b, os_a, os_b):
        wid = lax.axis_index("s") * nc + lax.axis_index("c")
        base = wid * b_per_w
        pltpu.sync_copy(idx_hbm.at[pl.ds(base, b_per_w)], idx_v)
        bufs = (buf_a, buf_b)
        gsems = (gs_a, gs_b)
        osems = (os_a, os_b)

        def start_gather(k):
            i = k % 2
            h = pltpu.make_async_copy(
                table_hbm.at[idx_v.at[pl.ds(offs[k], szs[k])]],
                bufs[i].at[pl.ds(0, szs[k])],
                gsems[i])
            h.start()
            return h

        def start_out(k):
            i = k % 2
            h = pltpu.make_async_copy(
                bufs[i].at[pl.ds(0, szs[k])],
                out_hbm.at[pl.ds(base + offs[k], szs[k])],
                osems[i])
            h.start()
            return h

        g = {0: start_gather(0)}
        o = {}
        for k in range(n):
            if k + 1 < n:
                if k - 1 in o:
                    o[k - 1].wait()
                g[k + 1] = start_gather(k + 1)
            g[k].wait()
            o[k] = start_out(k)
        o[n - 1].wait()
        if n >= 2:
            o[n - 2].wait()

    return gather_kernel


def _ln_body(gath_ref, seg_ref, pos_ref, segtab_ref, gamma_ref, beta_ref,
             out_ref):
    x = gath_ref[...]
    sid = seg_ref[...].astype(jnp.float32)
    pos = pos_ref[...]
    st = segtab_ref[...]
    seg = st[0][None, None, :] + sid[:, :, None] * (st[1] - st[0])[None, None, :]
    comb = x + pos[:, None, :] + seg
    mean = jnp.mean(comb, axis=-1, keepdims=True)
    var = jnp.mean(jnp.square(comb - mean), axis=-1, keepdims=True)
    xhat = (comb - mean) * lax.rsqrt(var + EPS)
    out_ref[...] = (xhat * gamma_ref[...][None, None, :]
                    + beta_ref[...][None, None, :])


def _ln_call(gath3, seg_ids, pos, seg_table, gamma, beta, b_blk=128):
    l, b, d = gath3.shape
    grid = (b // b_blk,)
    return pl.pallas_call(
        _ln_body,
        grid=grid,
        in_specs=[
            pl.BlockSpec((l, b_blk, d), lambda i: (0, i, 0)),
            pl.BlockSpec((l, b_blk), lambda i: (0, i)),
            pl.BlockSpec((l, d), lambda i: (0, 0)),
            pl.BlockSpec((2, d), lambda i: (0, 0)),
            pl.BlockSpec((d,), lambda i: (0,)),
            pl.BlockSpec((d,), lambda i: (0,)),
        ],
        out_specs=pl.BlockSpec((l, b_blk, d), lambda i: (0, i, 0)),
        out_shape=jax.ShapeDtypeStruct((l, b, d), jnp.float32),
    )(gath3, seg_ids, pos, seg_table, gamma, beta)


def kernel(tokens, segment_ids, token_table, pos_table, seg_table, gamma,
           beta):
    b, l = tokens.shape
    d = token_table.shape[1]
    flat = tokens.swapaxes(0, 1).reshape(-1).astype(jnp.int32)
    gathered = _make_sc_gather(b * l, d)(token_table, flat)
    gath3 = gathered.reshape(l, b, d)
    seg_t = segment_ids.swapaxes(0, 1).astype(jnp.int32)
    out = _ln_call(gath3, seg_t, pos_table[:l], seg_table, gamma, beta)
    return out.swapaxes(0, 1)
